# baseline (speedup 1.0000x reference)
import jax
import jax.numpy as jnp
from jax.experimental import pallas as pl


def _embed_kernel(table_ref, out_ref):
    out_ref[0, :, :] = table_ref[:, :]


def kernel(x, pos_table):
    maxlen = x.shape[-1]
    embed_dim = pos_table.shape[-1]
    return pl.pallas_call(
        _embed_kernel,
        out_shape=jax.ShapeDtypeStruct((1, maxlen, embed_dim), pos_table.dtype),
    )(pos_table[:maxlen])
